# baseline (speedup 1.0000x reference)
import jax
import jax.numpy as jnp
from jax.experimental import pallas as pl


_BATCH = 4096
_CLASSES = 1000
_DIM = 64
_INV_TEMP = 20.0


def _loss_kernel(e_ref, w_ref, lab_ref, loss_ref, comb_ref):
    e = e_ref[...]
    w = w_ref[...]
    en = e / (jnp.sqrt(jnp.sum(e * e, axis=1, keepdims=True)) + 1e-12)
    wn = w / (jnp.sqrt(jnp.sum(w * w, axis=1, keepdims=True)) + 1e-12)
    logits = jax.lax.dot_general(
        en, wn, (((1,), (1,)), ((), ())), preferred_element_type=jnp.float32
    ) * _INV_TEMP
    lse = jnp.log(jnp.sum(jnp.exp(logits), axis=1))
    labs = lab_ref[0, :]
    col = jax.lax.broadcasted_iota(jnp.int32, (_BATCH, _CLASSES), 1)
    tgt = jnp.sum(jnp.where(col == labs[:, None], logits, 0.0), axis=1)
    loss_ref[...] = (jnp.sum(lse - tgt) / _BATCH).reshape(1, 1)
    comb_ref[...] = jnp.broadcast_to(labs[None, :], (2, _BATCH))


def kernel(embeddings, labels, W):
    labs2 = labels.astype(jnp.int32).reshape(1, _BATCH)
    loss, comb = pl.pallas_call(
        _loss_kernel,
        out_shape=(
            jax.ShapeDtypeStruct((1, 1), jnp.float32),
            jax.ShapeDtypeStruct((2, _BATCH), jnp.int32),
        ),
    )(embeddings, W, labs2)
    combined_labels = comb.reshape(2 * _BATCH).astype(labels.dtype)
    return (loss[0, 0], combined_labels)


# speedup vs baseline: 1.0274x; 1.0274x over previous
import jax
import jax.numpy as jnp
from jax.experimental import pallas as pl


_BATCH = 4096
_CLASSES = 1000
_DIM = 64
_INV_TEMP = 20.0
_EXP2_SCALE = _INV_TEMP * 1.4426950408889634


def _loss_kernel(e_ref, w_ref, lab_ref, loss_ref, comb_ref):
    e = e_ref[...]
    w = w_ref[...]
    en = e * (1.0 / (jnp.sqrt(jnp.sum(e * e, axis=1, keepdims=True)) + 1e-12))
    wn = w * (1.0 / (jnp.sqrt(jnp.sum(w * w, axis=1, keepdims=True)) + 1e-12))
    cos = jax.lax.dot_general(
        en, wn, (((1,), (1,)), ((), ())), preferred_element_type=jnp.float32
    )
    lse = jnp.log(jnp.sum(jnp.exp2(cos * _EXP2_SCALE), axis=1))
    labs = lab_ref[0, :]
    col = jax.lax.broadcasted_iota(jnp.int32, (_BATCH, _CLASSES), 1)
    tgt = jnp.sum(jnp.where(col == labs[:, None], cos, 0.0), axis=1)
    loss_ref[...] = (jnp.sum(lse - _INV_TEMP * tgt) / _BATCH).reshape(1, 1)
    comb_ref[...] = jnp.broadcast_to(labs[None, :], (2, _BATCH))


def kernel(embeddings, labels, W):
    labs2 = labels.astype(jnp.int32).reshape(1, _BATCH)
    loss, comb = pl.pallas_call(
        _loss_kernel,
        out_shape=(
            jax.ShapeDtypeStruct((1, 1), jnp.float32),
            jax.ShapeDtypeStruct((2, _BATCH), jnp.int32),
        ),
    )(embeddings, W, labs2)
    combined_labels = comb.reshape(2 * _BATCH).astype(labels.dtype)
    return (loss[0, 0], combined_labels)


# speedup vs baseline: 1.0327x; 1.0052x over previous
import jax
import jax.numpy as jnp
from jax.experimental import pallas as pl


_BATCH = 4096
_CLASSES = 1000
_CPAD = 1024
_DIM = 64
_INV_TEMP = 20.0
_EXP2_SCALE = _INV_TEMP * 1.4426950408889634


def _loss_kernel(e_ref, w_ref, lab_ref, loss_ref, comb_ref):
    e = e_ref[...]
    w = w_ref[...]
    en = e * (1.0 / (jnp.sqrt(jnp.sum(e * e, axis=1, keepdims=True)) + 1e-12))
    wn = w * (1.0 / (jnp.sqrt(jnp.sum(w * w, axis=1, keepdims=True)) + 1e-12))
    wn = jnp.concatenate(
        [wn, jnp.zeros((_CPAD - _CLASSES, _DIM), jnp.float32)], axis=0
    )
    cos = jax.lax.dot_general(
        en, wn, (((1,), (1,)), ((), ())), preferred_element_type=jnp.float32
    )
    lse = jnp.log(jnp.sum(jnp.exp2(cos * _EXP2_SCALE), axis=1) - (_CPAD - _CLASSES))
    labs = lab_ref[0, :]
    col = jax.lax.broadcasted_iota(jnp.int32, (_BATCH, _CPAD), 1)
    tgt = jnp.sum(jnp.where(col == labs[:, None], cos, 0.0), axis=1)
    loss_ref[...] = (jnp.sum(lse - _INV_TEMP * tgt) / _BATCH).reshape(1, 1)
    comb_ref[...] = jnp.broadcast_to(labs[None, :], (2, _BATCH))


def kernel(embeddings, labels, W):
    labs2 = labels.astype(jnp.int32).reshape(1, _BATCH)
    loss, comb = pl.pallas_call(
        _loss_kernel,
        out_shape=(
            jax.ShapeDtypeStruct((1, 1), jnp.float32),
            jax.ShapeDtypeStruct((2, _BATCH), jnp.int32),
        ),
    )(embeddings, W, labs2)
    combined_labels = comb.reshape(2 * _BATCH).astype(labels.dtype)
    return (loss[0, 0], combined_labels)


# speedup vs baseline: 1.0361x; 1.0033x over previous
import jax
import jax.numpy as jnp
from jax.experimental import pallas as pl


_BATCH = 4096
_CLASSES = 1000
_CPAD = 1024
_DIM = 64
_INV_TEMP = 20.0
_EXP2_SCALE = _INV_TEMP * 1.4426950408889634


def _loss_kernel(e_ref, w_ref, lab_ref, loss_ref, comb_ref):
    e = e_ref[...]
    w = w_ref[...]
    en = e * (1.0 / (jnp.sqrt(jnp.sum(e * e, axis=1, keepdims=True)) + 1e-12))
    wn = w * (1.0 / (jnp.sqrt(jnp.sum(w * w, axis=1, keepdims=True)) + 1e-12))
    wn = jnp.concatenate(
        [wn, jnp.zeros((_CPAD - _CLASSES, _DIM), jnp.float32)], axis=0
    )
    cos = jax.lax.dot_general(
        en, wn, (((1,), (1,)), ((), ())), preferred_element_type=jnp.float32
    )
    lse = jnp.log(jnp.sum(jnp.exp2(cos * _EXP2_SCALE), axis=1) - (_CPAD - _CLASSES))
    labs = lab_ref[0, :]
    row = jax.lax.broadcasted_iota(jnp.int32, (_CPAD, _BATCH), 0)
    onehot_t = jnp.where(row == labs[None, :], 1.0, 0.0)
    z = jax.lax.dot_general(
        onehot_t, en, (((1,), (0,)), ((), ())), preferred_element_type=jnp.float32
    )
    tgt_sum = jnp.sum(z * wn)
    loss_ref[...] = ((jnp.sum(lse) - _INV_TEMP * tgt_sum) / _BATCH).reshape(1, 1)
    comb_ref[...] = jnp.broadcast_to(labs[None, :], (2, _BATCH))


def kernel(embeddings, labels, W):
    labs2 = labels.astype(jnp.int32).reshape(1, _BATCH)
    loss, comb = pl.pallas_call(
        _loss_kernel,
        out_shape=(
            jax.ShapeDtypeStruct((1, 1), jnp.float32),
            jax.ShapeDtypeStruct((2, _BATCH), jnp.int32),
        ),
    )(embeddings, W, labs2)
    combined_labels = comb.reshape(2 * _BATCH).astype(labels.dtype)
    return (loss[0, 0], combined_labels)
